# Initial kernel scaffold; baseline (speedup 1.0000x reference)
import jax
import jax.numpy as jnp
from jax.experimental import pallas as pl


def kernel(x, edge_index, W1, b1, W2, b2, W3, b3, g1, be1, g2, be2):
    raise NotImplementedError("write your pallas kernel here")


# speedup vs baseline: 8.1438x; 8.1438x over previous
import functools

import jax
import jax.numpy as jnp
from jax import lax
from jax.experimental import pallas as pl
from jax.experimental.pallas import tpu as pltpu
from jax.experimental.pallas import tpu_sc as plsc

_N = 10000
_E = 320000
_D = 128
_ALPHA = 0.5
_EPS = 1e-5

_NC = 2
_NS = 16
_NW = _NC * _NS
_CH = 128
_CPW = 79
_EPAD = _NW * _CPW * _CH
_ROWS = 10240
_RPT = _ROWS // _NS
_DUMMY = _N
_DEGW = 16
_ZR = 16

_mesh = plsc.VectorSubcoreMesh(core_axis_name="c", subcore_axis_name="s")


def _deg_body(rowp, out, rowv, hist):
    cid = lax.axis_index("c")
    sid = lax.axis_index("s")
    wid = sid * _NC + cid

    def zstep(i, carry):
        hist[pl.ds(i * 16, 16)] = jnp.zeros((16,), jnp.float32)
        return carry

    lax.fori_loop(0, _ROWS // 16, zstep, 0)
    pltpu.sync_copy(rowp.at[wid], rowv)
    ones = jnp.ones((16,), jnp.float32)

    def estep(c, carry):
        for k in range(_CH // 16):
            idx = rowv[c, pl.ds(k * 16, 16)]
            plsc.addupdate_scatter(hist, [idx], ones)
        return carry

    lax.fori_loop(0, _CPW, estep, 0)
    pltpu.sync_copy(hist, out.at[wid])


_deg_call = pl.kernel(
    _deg_body,
    out_type=jax.ShapeDtypeStruct((_NW, _ROWS), jnp.float32),
    mesh=_mesh,
    scratch_types=[
        pltpu.VMEM((_CPW, _CH), jnp.int32),
        pltpu.VMEM((_ROWS,), jnp.float32),
    ],
    compiler_params=pltpu.CompilerParams(needs_layout_passes=False),
)


def _agg_body(g, colp, rowp, out, colv, rowv, gbuf, zbuf, acc_sh, sem):
    cid = lax.axis_index("c")
    sid = lax.axis_index("s")
    wid = sid * _NC + cid
    for r in range(_ZR):
        for k in range(_D // 16):
            zbuf[r, pl.ds(k * 16, 16)] = jnp.zeros((16,), jnp.float32)

    def zstep(i, carry):
        pltpu.sync_copy(zbuf, acc_sh.at[pl.ds(sid * _RPT + i * _ZR, _ZR)])
        return carry

    lax.fori_loop(0, _RPT // _ZR, zstep, 0)
    pltpu.sync_copy(colp.at[wid], colv)
    pltpu.sync_copy(rowp.at[wid], rowv)
    plsc.subcore_barrier()

    def estep(c, carry):
        pltpu.async_copy(g.at[colv.at[c]], gbuf, sem).wait()
        pltpu.sync_copy(gbuf, acc_sh.at[rowv.at[c]], add=True)
        return carry

    lax.fori_loop(0, _CPW, estep, 0)
    plsc.subcore_barrier()
    pltpu.sync_copy(acc_sh.at[pl.ds(sid * _RPT, _RPT)],
                    out.at[cid, pl.ds(sid * _RPT, _RPT)])


_agg_call = pl.kernel(
    _agg_body,
    out_type=jax.ShapeDtypeStruct((_NC, _ROWS, _D), jnp.float32),
    mesh=_mesh,
    scratch_types=[
        pltpu.VMEM((_CPW, _CH), jnp.int32),
        pltpu.VMEM((_CPW, _CH), jnp.int32),
        pltpu.VMEM((_CH, _D), jnp.float32),
        pltpu.VMEM((_ZR, _D), jnp.float32),
        pltpu.VMEM_SHARED((_ROWS, _D), jnp.float32),
        pltpu.SemaphoreType.DMA,
    ],
)


def _prep_body(degp, x, dinv_ref, g_ref):
    deg = jnp.sum(degp[:, : _N], axis=0).reshape(_N, 1)
    dinv = jnp.where(deg > 0.0, lax.rsqrt(deg), 0.0)
    dinv_ref[...] = dinv
    g_ref[...] = x[...] * dinv


_prep_call = pl.pallas_call(
    _prep_body,
    out_shape=(
        jax.ShapeDtypeStruct((_N, 1), jnp.float32),
        jax.ShapeDtypeStruct((_N, _D), jnp.float32),
    ),
)


def _dense_body(h, aggp, dinv, W, b, gam, bet, hout, gout):
    dv = dinv[...]
    agg = aggp[0, : _N, :] + aggp[1, : _N, :]
    t = h[...] - _ALPHA * dv * agg
    z = jnp.dot(t, W[...], preferred_element_type=jnp.float32) + b[...]
    mu = jnp.mean(z, axis=0, keepdims=True)
    zc = z - mu
    var = jnp.mean(zc * zc, axis=0, keepdims=True)
    hn = jnp.maximum(zc * lax.rsqrt(var + _EPS) * gam[...] + bet[...], 0.0)
    hout[...] = hn
    gout[...] = hn * dv


_dense_call = pl.pallas_call(
    _dense_body,
    out_shape=(
        jax.ShapeDtypeStruct((_N, _D), jnp.float32),
        jax.ShapeDtypeStruct((_N, _D), jnp.float32),
    ),
)


def _final_body(h, aggp, dinv, W, b, out):
    agg = aggp[0, : _N, :] + aggp[1, : _N, :]
    t = h[...] - _ALPHA * dinv[...] * agg
    out[...] = jnp.dot(t, W[...], preferred_element_type=jnp.float32) + b[...]


_final_call = pl.pallas_call(
    _final_body,
    out_shape=jax.ShapeDtypeStruct((_N, _D), jnp.float32),
)


def kernel(x, edge_index, W1, b1, W2, b2, W3, b3, g1, be1, g2, be2):
    row = edge_index[0]
    col = edge_index[1]
    pad = _EPAD - _E
    rowp = jnp.concatenate(
        [row, jnp.full((pad,), _DUMMY, jnp.int32)]).reshape(_NW, _CPW, _CH)
    colp = jnp.concatenate(
        [col, jnp.zeros((pad,), jnp.int32)]).reshape(_NW, _CPW, _CH)

    degp = _deg_call(rowp)
    dinv, g = _prep_call(degp, x)

    aggp = _agg_call(g, colp, rowp)
    h, g = _dense_call(x, aggp, dinv, W1, b1.reshape(1, _D),
                       g1.reshape(1, _D), be1.reshape(1, _D))
    aggp = _agg_call(g, colp, rowp)
    h, g = _dense_call(h, aggp, dinv, W2, b2.reshape(1, _D),
                       g2.reshape(1, _D), be2.reshape(1, _D))
    aggp = _agg_call(g, colp, rowp)
    return _final_call(h, aggp, dinv, W3, b3.reshape(1, _D))
